# baseline (speedup 1.0000x reference)
import jax
import jax.numpy as jnp
from jax import lax
from jax.experimental import pallas as pl
from jax.experimental.pallas import tpu as pltpu
from jax.experimental.pallas import tpu_sc as plsc

_HIDDEN = 1024
_ROWS = 4096
_NC, _NS, _LANES = 2, 16, 16
_NW = _NC * _NS
_RPW = _ROWS // _NW
_CHUNK = 16
_NCHUNK = _RPW // _CHUNK
_NBUF = 3
_CWORDS = _CHUNK * _HIDDEN


def _body(feat_hbm, idx_hbm, table_hbm, out_hbm, idx_v, table_v, table_sh,
          fv0, fv1, fv2, si0, si1, si2, so0, so1, so2):
    bufs = (fv0, fv1, fv2)
    sin = (si0, si1, si2)
    sout = (so0, so1, so2)
    c_ax = lax.axis_index("c")
    s_ax = lax.axis_index("s")
    wid = s_ax * _NC + c_ax
    base = wid * _RPW

    def start_in(c, b):
        return pltpu.async_copy(
            feat_hbm.at[pl.ds((base + c * _CHUNK) * _HIDDEN, _CWORDS)],
            bufs[b], sin[b])

    def start_out(c, b):
        return pltpu.async_copy(
            bufs[b],
            out_hbm.at[pl.ds((base + c * _CHUNK) * _HIDDEN, _CWORDS)],
            sout[b])

    in_cp = [start_in(c, c) for c in range(_NBUF)]
    out_cp = [None] * _NBUF

    pltpu.sync_copy(idx_hbm.at[pl.ds(base, _RPW)], idx_v)

    @pl.when(s_ax == 0)
    def _():
        pltpu.sync_copy(table_hbm, table_sh)
    plsc.subcore_barrier()
    pltpu.sync_copy(table_sh, table_v)

    out_lanes = lax.iota(jnp.int32, 16) * _HIDDEN

    def compute(c, b):
        fv = bufs[b]
        ti0 = idx_v[pl.ds(c * _CHUNK, _LANES)] * _HIDDEN
        oi0 = out_lanes

        def col_blk(_, carry):
            ti, oi = carry
            for _u in range(16):
                tv = plsc.load_gather(table_v, [ti])
                plsc.addupdate_scatter(fv, [oi], tv)
                ti = ti + 1
                oi = oi + 1
            return ti, oi

        lax.fori_loop(0, _HIDDEN // 16, col_blk, (ti0, oi0))

    for c in range(_NCHUNK):
        b = c % _NBUF
        if c >= 1 and c + 2 < _NCHUNK:
            pb = (c + 2) % _NBUF
            out_cp[pb].wait()
            in_cp[pb] = start_in(c + 2, pb)
        in_cp[b].wait()
        compute(c, b)
        out_cp[b] = start_out(c, b)
    for b in range(_NBUF):
        out_cp[b].wait()


def kernel(features, frame_positions, temporal_pos_embedding_weight):
    b, f, h = features.shape
    feat_flat = features.reshape(b * f * h)
    idx = frame_positions.reshape(b * f)
    table_flat = temporal_pos_embedding_weight.reshape(-1)
    mesh = plsc.VectorSubcoreMesh(core_axis_name="c", subcore_axis_name="s")
    out = pl.kernel(
        _body,
        out_type=jax.ShapeDtypeStruct((b * f * h,), jnp.float32),
        mesh=mesh,
        compiler_params=pltpu.CompilerParams(needs_layout_passes=False),
        scratch_types=[
            pltpu.VMEM((_RPW,), jnp.int32),
            pltpu.VMEM((_HIDDEN * 64,), jnp.float32),
            pltpu.VMEM_SHARED((_HIDDEN * 64,), jnp.float32),
            pltpu.VMEM((_CWORDS,), jnp.float32),
            pltpu.VMEM((_CWORDS,), jnp.float32),
            pltpu.VMEM((_CWORDS,), jnp.float32),
            pltpu.SemaphoreType.DMA,
            pltpu.SemaphoreType.DMA,
            pltpu.SemaphoreType.DMA,
            pltpu.SemaphoreType.DMA,
            pltpu.SemaphoreType.DMA,
            pltpu.SemaphoreType.DMA,
        ],
    )(feat_flat, idx, table_flat)
    return out.reshape(b, f, h)


# speedup vs baseline: 3.2872x; 3.2872x over previous
import jax
import jax.numpy as jnp
from jax import lax
from jax.experimental import pallas as pl
from jax.experimental.pallas import tpu as pltpu
from jax.experimental.pallas import tpu_sc as plsc

_HIDDEN = 1024
_ROWS = 4096
_NC, _NS, _LANES = 2, 16, 16
_NW = _NC * _NS
_RPW = _ROWS // _NW
_CHUNK = 16
_NCHUNK = _RPW // _CHUNK
_NBUF = 3


def _body(feat_hbm, idx_hbm, table_hbm, out_hbm, idx_v, table_sh, table_v,
          fv0, fv1, fv2, sf0, sf1, sf2, so0, so1, so2):
    fbufs = (fv0, fv1, fv2)
    sf = (sf0, sf1, sf2)
    so = (so0, so1, so2)
    c_ax = lax.axis_index("c")
    s_ax = lax.axis_index("s")
    wid = s_ax * _NC + c_ax
    base = wid * _RPW

    pltpu.sync_copy(idx_hbm.at[pl.ds(base, _RPW)], idx_v)

    @pl.when(s_ax == 0)
    def _():
        pltpu.sync_copy(table_hbm, table_sh)
    plsc.subcore_barrier()
    pltpu.sync_copy(table_sh, table_v)

    def start_in(c, b):
        return pltpu.async_copy(
            feat_hbm.at[pl.ds(base + c * _CHUNK, _CHUNK)], fbufs[b], sf[b])

    def start_out(c, b):
        return pltpu.async_copy(
            fbufs[b], out_hbm.at[pl.ds(base + c * _CHUNK, _CHUNK)], so[b])

    in_cp = [start_in(c, c) for c in range(_NBUF)]
    out_cp = [None] * _NBUF

    def compute(c, b):
        fv = fbufs[b]
        iv = idx_v[pl.ds(c * _CHUNK, _CHUNK)]
        for r in range(_CHUNK):
            ridx = iv[r]

            def col_blk(j, carry, r=r, ridx=ridx):
                for u in range(4):
                    sl = pl.ds(j * 4 * _LANES + u * _LANES, _LANES)
                    plsc.addupdate(fv.at[r, sl], table_v[ridx, sl])
                return carry

            lax.fori_loop(0, _HIDDEN // (4 * _LANES), col_blk, 0,
                          unroll=False)

    for c in range(_NCHUNK):
        b = c % _NBUF
        if c >= 1 and c + 2 < _NCHUNK:
            pb = (c + 2) % _NBUF
            out_cp[pb].wait()
            in_cp[pb] = start_in(c + 2, pb)
        in_cp[b].wait()
        compute(c, b)
        out_cp[b] = start_out(c, b)
    for b in range(_NBUF):
        out_cp[b].wait()


def kernel(features, frame_positions, temporal_pos_embedding_weight):
    b, f, h = features.shape
    feat2 = features.reshape(b * f, h)
    idx = frame_positions.reshape(b * f)
    mesh = plsc.VectorSubcoreMesh(core_axis_name="c", subcore_axis_name="s")
    out = pl.kernel(
        _body,
        out_type=jax.ShapeDtypeStruct((b * f, h), jnp.float32),
        mesh=mesh,
        compiler_params=pltpu.CompilerParams(needs_layout_passes=False),
        scratch_types=[
            pltpu.VMEM((_RPW,), jnp.int32),
            pltpu.VMEM_SHARED((64, _HIDDEN), jnp.float32),
            pltpu.VMEM((64, _HIDDEN), jnp.float32),
            pltpu.VMEM((_CHUNK, _HIDDEN), jnp.float32),
            pltpu.VMEM((_CHUNK, _HIDDEN), jnp.float32),
            pltpu.VMEM((_CHUNK, _HIDDEN), jnp.float32),
            pltpu.SemaphoreType.DMA,
            pltpu.SemaphoreType.DMA,
            pltpu.SemaphoreType.DMA,
            pltpu.SemaphoreType.DMA,
            pltpu.SemaphoreType.DMA,
            pltpu.SemaphoreType.DMA,
        ],
    )(feat2, idx, temporal_pos_embedding_weight)
    return out.reshape(b, f, h)


# speedup vs baseline: 4.8391x; 1.4721x over previous
import jax
import jax.numpy as jnp
from jax import lax
from jax.experimental import pallas as pl
from jax.experimental.pallas import tpu as pltpu
from jax.experimental.pallas import tpu_sc as plsc

_HIDDEN = 1024
_ROWS = 4096
_NC, _NS, _LANES = 2, 16, 16
_NW = _NC * _NS
_RPW = _ROWS // _NW
_CHUNK = 16
_NCHUNK = _RPW // _CHUNK
_NBUF = 3


def _body(feat_hbm, idx_hbm, table_hbm, out_hbm, idx_v, table_sh, table_v,
          fv0, fv1, fv2, sf0, sf1, sf2, so0, so1, so2):
    fbufs = (fv0, fv1, fv2)
    sf = (sf0, sf1, sf2)
    so = (so0, so1, so2)
    c_ax = lax.axis_index("c")
    s_ax = lax.axis_index("s")
    wid = s_ax * _NC + c_ax
    base = wid * _RPW

    pltpu.sync_copy(idx_hbm.at[pl.ds(base, _RPW)], idx_v)

    @pl.when(s_ax == 0)
    def _():
        pltpu.sync_copy(table_hbm, table_sh)
    plsc.subcore_barrier()
    pltpu.sync_copy(table_sh, table_v)

    def start_in(c, b):
        return pltpu.async_copy(
            feat_hbm.at[pl.ds(base + c * _CHUNK, _CHUNK)], fbufs[b], sf[b])

    def start_out(c, b):
        return pltpu.async_copy(
            fbufs[b], out_hbm.at[pl.ds(base + c * _CHUNK, _CHUNK)], so[b])

    in_cp = [start_in(c, c) for c in range(_NBUF)]
    out_cp = [None] * _NBUF

    def compute(c, b):
        fv = fbufs[b]
        iv = idx_v[pl.ds(c * _CHUNK, _CHUNK)]
        for r0 in range(0, _CHUNK, 4):
            ridx = [iv[r0 + rr] for rr in range(4)]

            @plsc.parallel_loop(0, _HIDDEN // _LANES, unroll=4)
            def _(j, r0=r0, ridx=ridx):
                sl = pl.ds(j * _LANES, _LANES)
                for rr in range(4):
                    plsc.addupdate(fv.at[r0 + rr, sl],
                                   table_v[ridx[rr], sl])

    for c in range(_NCHUNK):
        b = c % _NBUF
        if c >= 1 and c + 2 < _NCHUNK:
            pb = (c + 2) % _NBUF
            out_cp[pb].wait()
            in_cp[pb] = start_in(c + 2, pb)
        in_cp[b].wait()
        compute(c, b)
        out_cp[b] = start_out(c, b)
    for b in range(_NBUF):
        out_cp[b].wait()


def kernel(features, frame_positions, temporal_pos_embedding_weight):
    b, f, h = features.shape
    feat2 = features.reshape(b * f, h)
    idx = frame_positions.reshape(b * f)
    mesh = plsc.VectorSubcoreMesh(core_axis_name="c", subcore_axis_name="s")
    out = pl.kernel(
        _body,
        out_type=jax.ShapeDtypeStruct((b * f, h), jnp.float32),
        mesh=mesh,
        compiler_params=pltpu.CompilerParams(needs_layout_passes=False),
        scratch_types=[
            pltpu.VMEM((_RPW,), jnp.int32),
            pltpu.VMEM_SHARED((64, _HIDDEN), jnp.float32),
            pltpu.VMEM((64, _HIDDEN), jnp.float32),
            pltpu.VMEM((_CHUNK, _HIDDEN), jnp.float32),
            pltpu.VMEM((_CHUNK, _HIDDEN), jnp.float32),
            pltpu.VMEM((_CHUNK, _HIDDEN), jnp.float32),
            pltpu.SemaphoreType.DMA,
            pltpu.SemaphoreType.DMA,
            pltpu.SemaphoreType.DMA,
            pltpu.SemaphoreType.DMA,
            pltpu.SemaphoreType.DMA,
            pltpu.SemaphoreType.DMA,
        ],
    )(feat2, idx, temporal_pos_embedding_weight)
    return out.reshape(b, f, h)


# speedup vs baseline: 11.1654x; 2.3073x over previous
import functools

import jax
import jax.numpy as jnp
from jax import lax
from jax.experimental import pallas as pl
from jax.experimental.pallas import tpu as pltpu
from jax.experimental.pallas import tpu_sc as plsc

_HIDDEN = 1024
_ROWS = 4096
_NC, _NS, _LANES = 2, 16, 16
_NW = _NC * _NS
_CHUNK = 16
_NBUF = 3
_K_SC = 0
_RB = 256


def _sc_body(feat_hbm, idx_hbm, table_hbm, out_hbm, idx_v, table_sh, table_v,
             fv0, fv1, fv2, sf0, sf1, sf2, so0, so1, so2):
    rpw = _K_SC // _NW
    nchunk = rpw // _CHUNK
    fbufs = (fv0, fv1, fv2)
    sf = (sf0, sf1, sf2)
    so = (so0, so1, so2)
    c_ax = lax.axis_index("c")
    s_ax = lax.axis_index("s")
    wid = s_ax * _NC + c_ax
    base = wid * rpw

    pltpu.sync_copy(idx_hbm.at[pl.ds(base, rpw)], idx_v)

    @pl.when(s_ax == 0)
    def _():
        pltpu.sync_copy(table_hbm, table_sh)
    plsc.subcore_barrier()
    pltpu.sync_copy(table_sh, table_v)

    def start_in(c, b):
        return pltpu.async_copy(
            feat_hbm.at[pl.ds(base + c * _CHUNK, _CHUNK)], fbufs[b], sf[b])

    def start_out(c, b):
        return pltpu.async_copy(
            fbufs[b], out_hbm.at[pl.ds(base + c * _CHUNK, _CHUNK)], so[b])

    in_cp = [start_in(c, c) for c in range(min(_NBUF, nchunk))]
    out_cp = [None] * _NBUF

    def compute(c, b):
        fv = fbufs[b]
        iv = idx_v[pl.ds(c * _CHUNK, _CHUNK)]
        for r0 in range(0, _CHUNK, 4):
            ridx = [iv[r0 + rr] for rr in range(4)]

            @plsc.parallel_loop(0, _HIDDEN // _LANES, unroll=4)
            def _(j, r0=r0, ridx=ridx):
                sl = pl.ds(j * _LANES, _LANES)
                for rr in range(4):
                    plsc.addupdate(fv.at[r0 + rr, sl],
                                   table_v[ridx[rr], sl])

    for c in range(nchunk):
        b = c % _NBUF
        if c >= 1 and c + 2 < nchunk:
            pb = (c + 2) % _NBUF
            out_cp[pb].wait()
            in_cp[pb] = start_in(c + 2, pb)
        in_cp[b].wait()
        compute(c, b)
        out_cp[b] = start_out(c, b)
    for b in range(min(_NBUF, nchunk)):
        out_cp[b].wait()


def _sc_call(feat2, idx, table):
    mesh = plsc.VectorSubcoreMesh(core_axis_name="c", subcore_axis_name="s")
    return pl.kernel(
        _sc_body,
        out_type=jax.ShapeDtypeStruct((_K_SC, _HIDDEN), jnp.float32),
        mesh=mesh,
        compiler_params=pltpu.CompilerParams(needs_layout_passes=False),
        scratch_types=[
            pltpu.VMEM((_K_SC // _NW,), jnp.int32),
            pltpu.VMEM_SHARED((64, _HIDDEN), jnp.float32),
            pltpu.VMEM((64, _HIDDEN), jnp.float32),
            pltpu.VMEM((_CHUNK, _HIDDEN), jnp.float32),
            pltpu.VMEM((_CHUNK, _HIDDEN), jnp.float32),
            pltpu.VMEM((_CHUNK, _HIDDEN), jnp.float32),
            pltpu.SemaphoreType.DMA,
            pltpu.SemaphoreType.DMA,
            pltpu.SemaphoreType.DMA,
            pltpu.SemaphoreType.DMA,
            pltpu.SemaphoreType.DMA,
            pltpu.SemaphoreType.DMA,
        ],
    )(feat2, idx, table)


def _tc_body(idx_ref, feat_ref, table_ref, out_ref):
    iv = idx_ref[0, 0, :]
    oh = (iv[:, None] == lax.broadcasted_iota(jnp.int32, (1, 64), 1)
          ).astype(jnp.float32)
    out_ref[...] = feat_ref[...] + jnp.dot(
        oh, table_ref[...], preferred_element_type=jnp.float32)


def _tc_call(feat2, idx3, table):
    n_tc = _ROWS - _K_SC
    k_blocks = _K_SC // _RB
    return pl.pallas_call(
        _tc_body,
        grid=(n_tc // _RB,),
        in_specs=[
            pl.BlockSpec((1, 1, _RB), lambda i: (k_blocks + i, 0, 0)),
            pl.BlockSpec((_RB, _HIDDEN), lambda i: (k_blocks + i, 0)),
            pl.BlockSpec((64, _HIDDEN), lambda i: (0, 0)),
        ],
        out_specs=pl.BlockSpec((_RB, _HIDDEN), lambda i: (i, 0)),
        out_shape=jax.ShapeDtypeStruct((n_tc, _HIDDEN), jnp.float32),
    )(idx3, feat2, table)


def kernel(features, frame_positions, temporal_pos_embedding_weight):
    b, f, h = features.shape
    feat2 = features.reshape(b * f, h)
    idx = frame_positions.reshape(b * f)
    idx3 = idx.reshape(_ROWS // _RB, 1, _RB)
    table = temporal_pos_embedding_weight
    parts = []
    if _K_SC > 0:
        parts.append(_sc_call(feat2, idx, table))
    if _K_SC < _ROWS:
        parts.append(_tc_call(feat2, idx3, table))
    out = parts[0] if len(parts) == 1 else jnp.concatenate(parts, axis=0)
    return out.reshape(b, f, h)
